# baseline (speedup 1.0000x reference)
import functools

import jax
import jax.numpy as jnp
from jax import lax
from jax.experimental import pallas as pl
from jax.experimental.pallas import tpu as pltpu
from jax.experimental.pallas import tpu_sc as plsc

NC, NS, L = 2, 16, 16
NW = NC * NS
B, S, D = 1024, 200, 128
SEQ_PER_W = B // NW
NBUF = 3


def _body(tok_hbm, idx_hbm, pos_hbm, out_hbm, pos_v, idx_v, rows_v, gsem, osem):
  wid = lax.axis_index("s") * NC + lax.axis_index("c")
  wbase = wid * (SEQ_PER_W * S)

  pltpu.sync_copy(pos_hbm, pos_v)
  pltpu.sync_copy(idx_hbm.at[pl.ds(wbase, SEQ_PER_W * S)], idx_v)

  def gather_desc(k, b):
    return pltpu.make_async_copy(
        tok_hbm.at[idx_v.at[pl.ds(k * S, S)]], rows_v.at[b], gsem.at[b])

  def scatter_desc(k, b):
    return pltpu.make_async_copy(
        rows_v.at[b], out_hbm.at[pl.ds(wbase + k * S, S)], osem.at[b])

  gather_desc(0, 0).start()
  gather_desc(1, 1).start()

  def step(k, carry):
    b = lax.rem(k, NBUF)

    @pl.when(k < SEQ_PER_W - 2)
    def _fire_ahead():
      bn = lax.rem(k + 2, NBUF)

      @pl.when(k >= 1)
      def _drain():
        scatter_desc(k - 1, bn).wait()

      gather_desc(k + 2, bn).start()

    gather_desc(k, b).wait()

    def row_body(r, c):
      for j in range(D // L):
        sl = pl.ds(j * L, L)
        plsc.addupdate(rows_v.at[b, r, sl], pos_v[r, sl])
      return c

    lax.fori_loop(0, S, row_body, 0)
    scatter_desc(k, b).start()
    return carry

  lax.fori_loop(0, SEQ_PER_W, step, 0)
  for k in (SEQ_PER_W - 3, SEQ_PER_W - 2, SEQ_PER_W - 1):
    scatter_desc(k, k % NBUF).wait()


@jax.jit
def _run(token_table, idx_flat, pos_table):
  mesh = plsc.VectorSubcoreMesh(
      core_axis_name="c", subcore_axis_name="s",
      num_cores=NC, num_subcores=NS)
  f = pl.kernel(
      _body,
      out_type=jax.ShapeDtypeStruct((B * S, D), jnp.float32),
      mesh=mesh,
      scratch_types=[
          pltpu.VMEM((S, D), jnp.float32),
          pltpu.VMEM((SEQ_PER_W * S,), jnp.int32),
          pltpu.VMEM((NBUF, S, D), jnp.float32),
          pltpu.SemaphoreType.DMA((NBUF,)),
          pltpu.SemaphoreType.DMA((NBUF,)),
      ],
  )
  return f(token_table, idx_flat, pos_table)


def kernel(inputs, token_table, pos_table):
  idx_flat = inputs.reshape(-1).astype(jnp.int32)
  out = _run(token_table, idx_flat, pos_table)
  return out.reshape(B, S, D)


# speedup vs baseline: 2.3759x; 2.3759x over previous
import functools

import jax
import jax.numpy as jnp
from jax import lax
from jax.experimental import pallas as pl
from jax.experimental.pallas import tpu as pltpu
from jax.experimental.pallas import tpu_sc as plsc

NC, NS, L = 2, 16, 16
NW = NC * NS
B, S, D = 1024, 200, 128
SEQ_PER_W = B // NW
NBUF = 3
NGRP = (SEQ_PER_W + NBUF - 1) // NBUF


def _body(tok_hbm, idx_hbm, pos_hbm, out_hbm, pos_v, idx_v, rows_v, gsem, osem):
  wid = lax.axis_index("s") * NC + lax.axis_index("c")
  wbase = wid * (SEQ_PER_W * S)

  pltpu.sync_copy(pos_hbm, pos_v)
  pltpu.sync_copy(idx_hbm.at[pl.ds(wbase, SEQ_PER_W * S)], idx_v)

  def gather_desc(k, b):
    return pltpu.make_async_copy(
        tok_hbm.at[idx_v.at[pl.ds(k * S, S)]], rows_v.at[b], gsem.at[b])

  def scatter_desc(k, b):
    return pltpu.make_async_copy(
        rows_v.at[b], out_hbm.at[pl.ds(wbase + k * S, S)], osem.at[b])

  gather_desc(0, 0).start()
  gather_desc(1, 1).start()

  def group(t, carry):
    for i in range(NBUF):
      k = t * NBUF + i

      @pl.when(k < SEQ_PER_W)
      def _slot():
        gather_desc(k, i).wait()

        def row_body(r, c):
          for j in range(D // L):
            sl = pl.ds(j * L, L)
            plsc.addupdate(rows_v.at[i, r, sl], pos_v[r, sl])
          return c

        lax.fori_loop(0, S, row_body, 0)

        bn = (i + 2) % NBUF

        @pl.when(k + 2 < SEQ_PER_W)
        def _fire_ahead():
          @pl.when(k >= 1)
          def _drain():
            scatter_desc(k - 1, bn).wait()

          gather_desc(k + 2, bn).start()

        scatter_desc(k, i).start()

    return carry

  lax.fori_loop(0, NGRP, group, 0)
  for k in (SEQ_PER_W - 3, SEQ_PER_W - 2, SEQ_PER_W - 1):
    scatter_desc(k, k % NBUF).wait()


@jax.jit
def _run(token_table, idx_flat, pos_table):
  mesh = plsc.VectorSubcoreMesh(
      core_axis_name="c", subcore_axis_name="s",
      num_cores=NC, num_subcores=NS)
  f = pl.kernel(
      _body,
      out_type=jax.ShapeDtypeStruct((B * S, D), jnp.float32),
      mesh=mesh,
      scratch_types=[
          pltpu.VMEM((S, D), jnp.float32),
          pltpu.VMEM((SEQ_PER_W * S,), jnp.int32),
          pltpu.VMEM((NBUF, S, D), jnp.float32),
          pltpu.SemaphoreType.DMA((NBUF,)),
          pltpu.SemaphoreType.DMA((NBUF,)),
      ],
  )
  return f(token_table, idx_flat, pos_table)


def kernel(inputs, token_table, pos_table):
  idx_flat = inputs.reshape(-1).astype(jnp.int32)
  out = _run(token_table, idx_flat, pos_table)
  return out.reshape(B, S, D)


# speedup vs baseline: 2.4444x; 1.0288x over previous
import functools

import jax
import jax.numpy as jnp
from jax import lax
from jax.experimental import pallas as pl
from jax.experimental.pallas import tpu as pltpu
from jax.experimental.pallas import tpu_sc as plsc

NC, NS, L = 2, 16, 16
NW = NC * NS
B, S, D = 1024, 200, 128
SEQ_PER_W = B // NW
NBUF = 3
NGRP = (SEQ_PER_W + NBUF - 1) // NBUF


def _body(tok_hbm, idx_hbm, pos_hbm, out_hbm, pos_v, idx_v, rows_v, gsem, osem):
  wid = lax.axis_index("s") * NC + lax.axis_index("c")
  wbase = wid * (SEQ_PER_W * S)

  pltpu.sync_copy(pos_hbm, pos_v)
  pltpu.sync_copy(idx_hbm.at[pl.ds(wbase, SEQ_PER_W * S)], idx_v)

  def gather_desc(k, b):
    return pltpu.make_async_copy(
        tok_hbm.at[idx_v.at[pl.ds(k * S, S)]], rows_v.at[b], gsem.at[b])

  def scatter_desc(k, b):
    return pltpu.make_async_copy(
        rows_v.at[b], out_hbm.at[pl.ds(wbase + k * S, S)], osem.at[b])

  gather_desc(0, 0).start()
  gather_desc(1, 1).start()

  def group(t, carry):
    for i in range(NBUF):
      k = t * NBUF + i

      @pl.when(k < SEQ_PER_W)
      def _slot():
        gather_desc(k, i).wait()

        @functools.partial(plsc.parallel_loop, 0, S, unroll=4)
        def _add(r):
          for j in range(D // L):
            sl = pl.ds(j * L, L)
            plsc.addupdate(rows_v.at[i, r, sl], pos_v[r, sl])

        bn = (i + 2) % NBUF

        @pl.when(k + 2 < SEQ_PER_W)
        def _fire_ahead():
          @pl.when(k >= 1)
          def _drain():
            scatter_desc(k - 1, bn).wait()

          gather_desc(k + 2, bn).start()

        scatter_desc(k, i).start()

    return carry

  lax.fori_loop(0, NGRP, group, 0)
  for k in (SEQ_PER_W - 3, SEQ_PER_W - 2, SEQ_PER_W - 1):
    scatter_desc(k, k % NBUF).wait()


@jax.jit
def _run(token_table, idx_flat, pos_table):
  mesh = plsc.VectorSubcoreMesh(
      core_axis_name="c", subcore_axis_name="s",
      num_cores=NC, num_subcores=NS)
  f = pl.kernel(
      _body,
      out_type=jax.ShapeDtypeStruct((B * S, D), jnp.float32),
      mesh=mesh,
      scratch_types=[
          pltpu.VMEM((S, D), jnp.float32),
          pltpu.VMEM((SEQ_PER_W * S,), jnp.int32),
          pltpu.VMEM((NBUF, S, D), jnp.float32),
          pltpu.SemaphoreType.DMA((NBUF,)),
          pltpu.SemaphoreType.DMA((NBUF,)),
      ],
  )
  return f(token_table, idx_flat, pos_table)


def kernel(inputs, token_table, pos_table):
  idx_flat = inputs.reshape(-1).astype(jnp.int32)
  out = _run(token_table, idx_flat, pos_table)
  return out.reshape(B, S, D)
